# Initial kernel scaffold; baseline (speedup 1.0000x reference)
import jax
import jax.numpy as jnp
from jax.experimental import pallas as pl


def kernel(x, edge_index, edge_weight, diff_edge, diff_weight, params):
    raise NotImplementedError("write your pallas kernel here")


# speedup vs baseline: 3.1734x; 3.1734x over previous
import functools

import jax
import jax.numpy as jnp
from jax import lax
from jax.experimental import pallas as pl
from jax.experimental.pallas import tpu as pltpu
from jax.experimental.pallas import tpu_sc as plsc

NC = 2
NS = 16
CH = 128


def _mm2_body(x_ref, w_ref, o_ref):
    o_ref[0] = jnp.dot(x_ref[...], w_ref[0], preferred_element_type=jnp.float32)


def _mm2(x, wstack):
    n, d = x.shape
    h = wstack.shape[2]
    return pl.pallas_call(
        _mm2_body,
        grid=(2,),
        in_specs=[pl.BlockSpec((n, d), lambda c: (0, 0)),
                  pl.BlockSpec((1, d, h), lambda c: (c, 0, 0))],
        out_specs=pl.BlockSpec((1, n, h), lambda c: (c, 0, 0)),
        out_shape=jax.ShapeDtypeStruct((2, n, h), jnp.float32),
    )(x, wstack)


def _bn(h, g, be):
    m = jnp.mean(h, axis=0)
    v = jnp.mean((h - m) ** 2, axis=0)
    return (h - m) * lax.rsqrt(v + 1e-5) * g + be


def _bnmm_body(a_ref, b_ref, g_ref, be_ref, w_ref, o_ref):
    h = _bn(a_ref[0] + b_ref[0, 0], g_ref[0, 0], be_ref[0, 0])
    o_ref[0] = jnp.dot(jnp.maximum(h, 0.0), w_ref[0],
                       preferred_element_type=jnp.float32)


def _bnmm(agg, b, g, be, wstack):
    _, n, h = agg.shape
    return pl.pallas_call(
        _bnmm_body,
        grid=(2,),
        in_specs=[pl.BlockSpec((1, n, h), lambda c: (c, 0, 0)),
                  pl.BlockSpec((1, 1, h), lambda c: (c, 0, 0)),
                  pl.BlockSpec((1, 1, h), lambda c: (c, 0, 0)),
                  pl.BlockSpec((1, 1, h), lambda c: (c, 0, 0)),
                  pl.BlockSpec((1, h, h), lambda c: (c, 0, 0))],
        out_specs=pl.BlockSpec((1, n, h), lambda c: (c, 0, 0)),
        out_shape=jax.ShapeDtypeStruct((2, n, h), jnp.float32),
    )(agg, b, g, be, wstack)


def _final_body(a_ref, b_ref, g_ref, be_ref, wc_ref, bc_ref, o_ref):
    s0 = _bn(a_ref[0] + b_ref[0, 0], g_ref[0, 0], be_ref[0, 0])
    s1 = _bn(a_ref[1] + b_ref[1, 0], g_ref[1, 0], be_ref[1, 0])
    o_ref[...] = (jnp.dot(s0 + s1, wc_ref[...],
                          preferred_element_type=jnp.float32) + bc_ref[0])


def _final(agg, b, g, be, wc, bc):
    _, n, h = agg.shape
    out = wc.shape[1]
    return pl.pallas_call(
        _final_body,
        out_shape=jax.ShapeDtypeStruct((n, out), jnp.float32),
    )(agg, b, g, be, wc, bc.reshape(1, out))


_GDN = lax.GatherDimensionNumbers(
    offset_dims=(), collapsed_slice_dims=(0,), start_index_map=(0,))


def _bcast_lane(wv, l):
    idx = jnp.full((16, 1), l, jnp.int32)
    return lax.gather(wv, idx, _GDN, slice_sizes=(1,),
                      mode=lax.GatherScatterMode.PROMISE_IN_BOUNDS)

def _make_gconv(n, h, ept, ep):
    nch = ept // CH
    q = (n // (8 * NS)) * 8
    r = n - NS * q
    assert 0 <= r <= CH and r % 8 == 0
    kq, kr = divmod(q, CH)
    nlan = h // 16
    mesh = plsc.VectorSubcoreMesh(core_axis_name="c", subcore_axis_name="s")

    def body(hh, srcf, dstf, ewf, outf, acc, sidx, didx, ewv, rows, gsem):
        cid = lax.axis_index("c")
        sid = lax.axis_index("s")

        @pl.loop(0, CH)
        def _(i):
            for j in range(nlan):
                rows[0, i, pl.ds(j * 16, 16)] = jnp.zeros((16,), jnp.float32)

        row0 = sid * q
        for k in range(kq):
            pltpu.sync_copy(rows.at[0], acc.at[pl.ds(row0 + k * CH, CH)])
        if kr:
            pltpu.sync_copy(rows.at[0, pl.ds(0, kr)],
                            acc.at[pl.ds(row0 + kq * CH, kr)])
        if r:
            @pl.when(sid == 0)
            def _():
                pltpu.sync_copy(rows.at[0, pl.ds(0, r)],
                                acc.at[pl.ds(NS * q, r)])
        plsc.subcore_barrier()

        ebase = cid * ep + sid * ept

        @pl.loop(0, nch)
        def _(g):
            base = ebase + g * CH
            pltpu.sync_copy(srcf.at[pl.ds(base, CH)], sidx.at[0])
            pltpu.sync_copy(dstf.at[pl.ds(base, CH)], didx.at[0])
            pltpu.sync_copy(ewf.at[pl.ds(base, CH)], ewv.at[0])
            pltpu.async_copy(hh.at[sidx.at[0]], rows.at[0], gsem).wait()

            @pl.loop(0, CH // 16)
            def _(gg):
                wv = ewv[0, pl.ds(gg * 16, 16)]
                for l in range(16):
                    w = _bcast_lane(wv, l)
                    e = gg * 16 + l
                    for j in range(nlan):
                        rows[0, e, pl.ds(j * 16, 16)] = (
                            rows[0, e, pl.ds(j * 16, 16)] * w)

            pltpu.sync_copy(rows.at[0], acc.at[didx.at[0]], add=True)

        plsc.subcore_barrier()

        out0 = cid * n + row0
        for k in range(kq):
            pltpu.sync_copy(acc.at[pl.ds(row0 + k * CH, CH)], rows.at[0])
            pltpu.sync_copy(rows.at[0], outf.at[pl.ds(out0 + k * CH, CH)])
        if kr:
            pltpu.sync_copy(acc.at[pl.ds(row0 + kq * CH, kr)],
                            rows.at[0, pl.ds(0, kr)])
            pltpu.sync_copy(rows.at[0, pl.ds(0, kr)],
                            outf.at[pl.ds(out0 + kq * CH, kr)])
        if r:
            @pl.when(sid == NS - 1)
            def _():
                pltpu.sync_copy(acc.at[pl.ds(NS * q, r)],
                                rows.at[0, pl.ds(0, r)])
                pltpu.sync_copy(rows.at[0, pl.ds(0, r)],
                                outf.at[pl.ds(cid * n + NS * q, r)])

    return pl.kernel(
        body,
        out_type=jax.ShapeDtypeStruct((2 * n, h), jnp.float32),
        mesh=mesh,
        scratch_types=[
            pltpu.VMEM_SHARED((n, h), jnp.float32),
            pltpu.VMEM((1, CH), jnp.int32),
            pltpu.VMEM((1, CH), jnp.int32),
            pltpu.VMEM((1, CH), jnp.float32),
            pltpu.VMEM((1, CH, h), jnp.float32),
            pltpu.SemaphoreType.DMA,
        ],
    )


def kernel(x, edge_index, edge_weight, diff_edge, diff_weight, params):
    n, d = x.shape
    h = params['a_W1'].shape[1]
    e = edge_weight.shape[0]

    ept = -(-e // (NS * CH)) * CH
    ep = ept * NS
    padn = ep - e

    def pad(a):
        return jnp.pad(a, (0, padn))

    srcf = jnp.concatenate([pad(edge_index[0]), pad(diff_edge[0]) + n])
    dstf = jnp.concatenate([pad(edge_index[1]), pad(diff_edge[1])])
    ewf = jnp.concatenate([pad(edge_weight), pad(diff_weight)])

    p = params
    W1 = jnp.stack([p['a_W1'], p['d_W1']])
    b1 = jnp.stack([p['a_b1'], p['d_b1']]).reshape(2, 1, h)
    g1 = jnp.stack([p['a_g1'], p['d_g1']]).reshape(2, 1, h)
    be1 = jnp.stack([p['a_be1'], p['d_be1']]).reshape(2, 1, h)
    W2 = jnp.stack([p['a_W2'], p['d_W2']])
    b2 = jnp.stack([p['a_b2'], p['d_b2']]).reshape(2, 1, h)
    g2 = jnp.stack([p['a_g2'], p['d_g2']]).reshape(2, 1, h)
    be2 = jnp.stack([p['a_be2'], p['d_be2']]).reshape(2, 1, h)

    gconv = _make_gconv(n, h, ept, ep)

    hh1 = _mm2(x, W1)
    agg1 = gconv(hh1.reshape(2 * n, h), srcf, dstf, ewf).reshape(2, n, h)
    hh2 = _bnmm(agg1, b1, g1, be1, W2)
    agg2 = gconv(hh2.reshape(2 * n, h), srcf, dstf, ewf).reshape(2, n, h)
    return _final(agg2, b2, g2, be2, p['Wc'], p['bc'])
